# baseline (speedup 1.0000x reference)
import jax
import jax.numpy as jnp
import numpy as np
from jax import lax
from jax.experimental import pallas as pl
from jax.experimental.pallas import tpu as pltpu
from jax.experimental.pallas import tpu_sc as plsc

KK = 8
B, H, W, C = 8, 224, 224, 192
HW = H * W
G = 32
NJ = H // G
NBLK = NJ * W
L = 16
NCAND = KK * G
NW = 32
TPW = (B * C) // NW
XCR = B * H * C
HSTRIDE = C * 8
ROWS16 = 2 * XCR * 8

_NEG_INF = float("-inf")


def _stage1_body(x_ref, mx_ref, mn_ref, xc_ref):
    x = x_ref[...]
    mx_ref[...] = jnp.max(x, axis=1, keepdims=True)
    mn_ref[...] = jnp.min(x, axis=1, keepdims=True)
    xc_ref[0] = x[0, :, :, 0:128].reshape(G * C, 128)
    xc_ref[1] = x[0, :, :, 96:224].reshape(G * C, 128)


def _transpose_body(mx_ref, mn_ref, mxt_ref, mnt_ref):
    mxt_ref[...] = jnp.transpose(mx_ref[...], (0, 2, 1, 3))
    mnt_ref[...] = jnp.transpose(mn_ref[...], (0, 2, 1, 3))


def _merge16(rv, ri, v, vi):
    sv, si = plsc.sort_key_val(v, vi, descending=True)
    keep = rv >= sv
    nv = jnp.where(keep, rv, sv)
    ni = jnp.where(keep, ri, si)
    srt = plsc.sort_key_val(nv, ni)
    return srt[0], srt[1]


def _scan_row(row_ref, nsteps, negate):
    iota = lax.iota(jnp.int32, L)

    def step(i, carry):
        rv, ri, thr = carry
        v = row_ref[pl.ds(i * L, L)]
        if negate:
            v = -v
        hit = jnp.any(v > thr)

        def do_merge(_):
            nrv, nri = _merge16(rv, ri, v, iota + i * L)
            return nrv, nri, jnp.min(nrv)

        def skip(_):
            return rv, ri, thr

        return lax.cond(hit, do_merge, skip, 0)

    rv0 = jnp.full((L,), _NEG_INF, jnp.float32)
    ri0 = jnp.zeros((L,), jnp.int32)
    rv, ri, _ = lax.fori_loop(0, nsteps, step,
                              (rv0, ri0, jnp.float32(_NEG_INF)))
    return rv, ri


def _sc_body(x2d, mxt, mnt, boff, pblk, poff, outf,
             mrow_v, nrow_v, boff_v, bb_v, bl_v, idx_v, rows_v,
             pblk_v, poff_v, out_v, sem):
    cid = lax.axis_index("c")
    sid = lax.axis_index("s")
    w = sid * 2 + cid
    b = w // 4
    c0 = 48 * (w % 4)
    iota = lax.iota(jnp.int32, L)

    pltpu.sync_copy(boff, boff_v)
    pltpu.sync_copy(pblk, pblk_v)
    pltpu.sync_copy(poff, poff_v)

    def task(tloc, _):
        c = c0 + tloc
        t = b * C + c
        pltpu.sync_copy(mxt.at[pl.ds(t * NBLK, NBLK)], mrow_v)
        pltpu.sync_copy(mnt.at[pl.ds(t * NBLK, NBLK)], nrow_v)
        rowbase = b * (H * C * 8) + c * 8

        def side(row_ref, negate, lane_off):
            rv, ri = _scan_row(row_ref, NBLK // L, negate)
            bb_v[...] = plsc.load_gather(boff_v, [ri]) + rowbase
            bl_v[...] = jnp.bitwise_and(ri, L - 1)

            def build(i, _b):
                pb = pblk_v[pl.ds(i * L, L)]
                po = poff_v[pl.ds(i * L, L)]
                bv = plsc.load_gather(bb_v, [pb])
                idx_v[pl.ds(i * L, L)] = bv + po
                return 0

            lax.fori_loop(0, NCAND // L, build, 0)

            cp0 = pltpu.async_copy(x2d.at[idx_v.at[pl.ds(0, 128)]],
                                   rows_v.at[pl.ds(0, 128)], sem)
            cp1 = pltpu.async_copy(x2d.at[idx_v.at[pl.ds(128, 128)]],
                                   rows_v.at[pl.ds(128, 128)], sem)
            cp0.wait()
            cp1.wait()

            def cstep(i, rv2):
                pb = pblk_v[pl.ds(i * L, L)]
                lv = plsc.load_gather(bl_v, [pb])
                cv = plsc.load_gather(rows_v, [iota + i * L, lv])
                if negate:
                    cv = -cv
                nv, _nv2 = _merge16(rv2, rv2, cv, cv)
                return nv

            rv2 = lax.fori_loop(0, NCAND // L, cstep,
                                jnp.full((L,), _NEG_INF, jnp.float32))

            best_desc = lax.rev(rv2, (0,))
            if negate:
                vals = -best_desc
            else:
                vals = best_desc
            plsc.store_scatter(out_v, [iota + (16 * tloc + lane_off)], vals,
                               mask=iota < 8)

        side(mrow_v, False, 0)
        side(nrow_v, True, 8)
        return 0

    lax.fori_loop(0, TPW, task, 0)
    pltpu.sync_copy(out_v, outf.at[pl.ds(w * (TPW * 16), TPW * 16)])


def _make_tables():
    blk = np.arange(NBLK)
    jj, ww = blk // W, blk % W
    sel = (ww >= 128).astype(np.int64)
    wadj = ww - 96 * sel
    boff = (sel * (XCR * 8) + jj * G * HSTRIDE + wadj // L).astype(np.int32)
    j = np.arange(NCAND)
    pblk = (8 + j // G).astype(np.int32)
    poff = ((j % G) * HSTRIDE).astype(np.int32)
    return jnp.asarray(boff), jnp.asarray(pblk), jnp.asarray(poff)


@jax.jit
def kernel(inputs):
    xt = jnp.transpose(inputs, (0, 1, 3, 2))

    mx, mn, xcopy = pl.pallas_call(
        _stage1_body,
        grid=(B, NJ),
        in_specs=[pl.BlockSpec((1, G, C, W), lambda b, j: (b, j, 0, 0))],
        out_specs=[
            pl.BlockSpec((1, 1, C, W), lambda b, j: (b, j, 0, 0)),
            pl.BlockSpec((1, 1, C, W), lambda b, j: (b, j, 0, 0)),
            pl.BlockSpec((2, G * C, 128), lambda b, j: (0, b * NJ + j, 0)),
        ],
        out_shape=[
            jax.ShapeDtypeStruct((B, NJ, C, W), jnp.float32),
            jax.ShapeDtypeStruct((B, NJ, C, W), jnp.float32),
            jax.ShapeDtypeStruct((2, XCR, 128), jnp.float32),
        ],
    )(xt)

    mxt, mnt = pl.pallas_call(
        _transpose_body,
        grid=(B,),
        in_specs=[pl.BlockSpec((1, NJ, C, W), lambda b: (b, 0, 0, 0)),
                  pl.BlockSpec((1, NJ, C, W), lambda b: (b, 0, 0, 0))],
        out_specs=[pl.BlockSpec((1, C, NJ, W), lambda b: (b, 0, 0, 0)),
                   pl.BlockSpec((1, C, NJ, W), lambda b: (b, 0, 0, 0))],
        out_shape=[jax.ShapeDtypeStruct((B, C, NJ, W), jnp.float32)] * 2,
    )(mx, mn)

    x2d = xcopy.reshape(ROWS16, L)
    boff, pblk, poff = _make_tables()

    mesh = plsc.VectorSubcoreMesh(core_axis_name="c", subcore_axis_name="s",
                                  num_cores=2, num_subcores=16)
    outf = pl.kernel(
        _sc_body,
        out_type=jax.ShapeDtypeStruct((B * C * 2 * KK,), jnp.float32),
        mesh=mesh,
        compiler_params=pltpu.CompilerParams(needs_layout_passes=False,
                                             use_tc_tiling_on_sc=False),
        scratch_types=[
            pltpu.VMEM((NBLK,), jnp.float32),
            pltpu.VMEM((NBLK,), jnp.float32),
            pltpu.VMEM((NBLK,), jnp.int32),
            pltpu.VMEM((L,), jnp.int32),
            pltpu.VMEM((L,), jnp.int32),
            pltpu.VMEM((NCAND,), jnp.int32),
            pltpu.VMEM((NCAND, L), jnp.float32),
            pltpu.VMEM((NCAND,), jnp.int32),
            pltpu.VMEM((NCAND,), jnp.int32),
            pltpu.VMEM((TPW * 16,), jnp.float32),
            pltpu.SemaphoreType.DMA,
        ],
    )(x2d, mxt.reshape(B * C * NBLK), mnt.reshape(B * C * NBLK),
      boff, pblk, poff)

    return outf.reshape(B, 2 * KK * C)


# speedup vs baseline: 1.8498x; 1.8498x over previous
import jax
import jax.numpy as jnp
import numpy as np
from jax import lax
from jax.experimental import pallas as pl
from jax.experimental.pallas import tpu as pltpu
from jax.experimental.pallas import tpu_sc as plsc

KK = 8
B, H, W, C = 8, 224, 224, 192
HW = H * W
G = 32
NJ = H // G
NBLK = NJ * W
L = 16
NCAND = KK * G
NW = 32
BH = 4
TPW = (BH * C) // NW
XCR = BH * H * C
HSTRIDE = C * 8
ROWS16 = 2 * XCR * 8

_NEG_INF = float("-inf")


def _stage1_body(x_ref, mx_ref, mn_ref, xc_ref):
    x = x_ref[...]
    mx_ref[...] = jnp.max(x, axis=1, keepdims=True)
    mn_ref[...] = jnp.min(x, axis=1, keepdims=True)
    xc_ref[0] = x[0, :, :, 0:128].reshape(G * C, 128)
    xc_ref[1] = x[0, :, :, 96:224].reshape(G * C, 128)


def _transpose_body(mx_ref, mn_ref, mxt_ref, mnt_ref):
    mxt_ref[...] = jnp.transpose(mx_ref[...], (0, 2, 1, 3))
    mnt_ref[...] = jnp.transpose(mn_ref[...], (0, 2, 1, 3))


def _merge16(rv, ri, v, vi):
    sv, si = plsc.sort_key_val(v, vi, descending=True)
    keep = rv >= sv
    nv = jnp.where(keep, rv, sv)
    ni = jnp.where(keep, ri, si)
    srt = plsc.sort_key_val(nv, ni)
    return srt[0], srt[1]


def _scan_row(row_ref, nsteps, negate):
    iota = lax.iota(jnp.int32, L)

    def step(i, carry):
        rv, ri = carry
        v = row_ref[pl.ds(i * L, L)]
        if negate:
            v = -v
        return _merge16(rv, ri, v, iota + i * L)

    rv0 = jnp.full((L,), _NEG_INF, jnp.float32)
    ri0 = jnp.zeros((L,), jnp.int32)
    rv, ri = lax.fori_loop(0, nsteps, step, (rv0, ri0))
    return rv, ri


def _sc_body(x2d, mxt, mnt, boff, pblk, poff, outf,
             mrow_v, nrow_v, boff_v, bb_v, bl_v, idx_v, rows_v,
             pblk_v, poff_v, out_v, sem):
    cid = lax.axis_index("c")
    sid = lax.axis_index("s")
    w = sid * 2 + cid
    b = w // 8
    c0 = 24 * (w % 8)
    iota = lax.iota(jnp.int32, L)

    pltpu.sync_copy(boff, boff_v)
    pltpu.sync_copy(pblk, pblk_v)
    pltpu.sync_copy(poff, poff_v)

    def task(tloc, _):
        c = c0 + tloc
        t = b * C + c
        pltpu.sync_copy(mxt.at[pl.ds(t * NBLK, NBLK)], mrow_v)
        pltpu.sync_copy(mnt.at[pl.ds(t * NBLK, NBLK)], nrow_v)
        rowbase = b * (H * C * 8) + c * 8

        def side(row_ref, negate, lane_off):
            rv, ri = _scan_row(row_ref, NBLK // L, negate)
            bb_v[...] = plsc.load_gather(boff_v, [ri]) + rowbase
            bl_v[...] = jnp.bitwise_and(ri, L - 1)

            def build(i, _b):
                pb = pblk_v[pl.ds(i * L, L)]
                po = poff_v[pl.ds(i * L, L)]
                bv = plsc.load_gather(bb_v, [pb])
                idx_v[pl.ds(i * L, L)] = bv + po
                return 0

            lax.fori_loop(0, NCAND // L, build, 0)

            cp0 = pltpu.async_copy(x2d.at[idx_v.at[pl.ds(0, 128)]],
                                   rows_v.at[pl.ds(0, 128)], sem)
            cp1 = pltpu.async_copy(x2d.at[idx_v.at[pl.ds(128, 128)]],
                                   rows_v.at[pl.ds(128, 128)], sem)
            cp0.wait()
            cp1.wait()

            def cstep(i, rv2):
                pb = pblk_v[pl.ds(i * L, L)]
                lv = plsc.load_gather(bl_v, [pb])
                cv = plsc.load_gather(rows_v, [iota + i * L, lv])
                if negate:
                    cv = -cv
                nv, _nv2 = _merge16(rv2, rv2, cv, cv)
                return nv

            rv2 = lax.fori_loop(0, NCAND // L, cstep,
                                jnp.full((L,), _NEG_INF, jnp.float32))

            best_desc = lax.rev(rv2, (0,))
            if negate:
                vals = -best_desc
            else:
                vals = best_desc
            plsc.store_scatter(out_v, [iota + (16 * tloc + lane_off)], vals,
                               mask=iota < 8)

        side(mrow_v, False, 0)
        side(nrow_v, True, 8)
        return 0

    lax.fori_loop(0, TPW, task, 0)
    pltpu.sync_copy(out_v, outf.at[pl.ds(w * (TPW * 16), TPW * 16)])


def _make_tables():
    blk = np.arange(NBLK)
    jj, ww = blk // W, blk % W
    sel = (ww >= 128).astype(np.int64)
    wadj = ww - 96 * sel
    boff = (sel * (XCR * 8) + jj * G * HSTRIDE + wadj // L).astype(np.int32)
    j = np.arange(NCAND)
    pblk = (8 + j // G).astype(np.int32)
    poff = ((j % G) * HSTRIDE).astype(np.int32)
    return jnp.asarray(boff), jnp.asarray(pblk), jnp.asarray(poff)


def _wave(xt, b0, tables):
    boff, pblk, poff = tables

    mx, mn, xcopy = pl.pallas_call(
        _stage1_body,
        grid=(BH, NJ),
        in_specs=[pl.BlockSpec((1, G, C, W), lambda b, j: (b + b0, j, 0, 0))],
        out_specs=[
            pl.BlockSpec((1, 1, C, W), lambda b, j: (b, j, 0, 0)),
            pl.BlockSpec((1, 1, C, W), lambda b, j: (b, j, 0, 0)),
            pl.BlockSpec((2, G * C, 128), lambda b, j: (0, b * NJ + j, 0)),
        ],
        out_shape=[
            jax.ShapeDtypeStruct((BH, NJ, C, W), jnp.float32),
            jax.ShapeDtypeStruct((BH, NJ, C, W), jnp.float32),
            jax.ShapeDtypeStruct((2, XCR, 128), jnp.float32),
        ],
    )(xt)

    mxt, mnt = pl.pallas_call(
        _transpose_body,
        grid=(BH,),
        in_specs=[pl.BlockSpec((1, NJ, C, W), lambda b: (b, 0, 0, 0)),
                  pl.BlockSpec((1, NJ, C, W), lambda b: (b, 0, 0, 0))],
        out_specs=[pl.BlockSpec((1, C, NJ, W), lambda b: (b, 0, 0, 0)),
                   pl.BlockSpec((1, C, NJ, W), lambda b: (b, 0, 0, 0))],
        out_shape=[jax.ShapeDtypeStruct((BH, C, NJ, W), jnp.float32)] * 2,
    )(mx, mn)

    x2d = xcopy.reshape(ROWS16, L)

    mesh = plsc.VectorSubcoreMesh(core_axis_name="c", subcore_axis_name="s",
                                  num_cores=2, num_subcores=16)
    return pl.kernel(
        _sc_body,
        out_type=jax.ShapeDtypeStruct((BH * C * 2 * KK,), jnp.float32),
        mesh=mesh,
        compiler_params=pltpu.CompilerParams(needs_layout_passes=False,
                                             use_tc_tiling_on_sc=False),
        scratch_types=[
            pltpu.VMEM((NBLK,), jnp.float32),
            pltpu.VMEM((NBLK,), jnp.float32),
            pltpu.VMEM((NBLK,), jnp.int32),
            pltpu.VMEM((L,), jnp.int32),
            pltpu.VMEM((L,), jnp.int32),
            pltpu.VMEM((NCAND,), jnp.int32),
            pltpu.VMEM((NCAND, L), jnp.float32),
            pltpu.VMEM((NCAND,), jnp.int32),
            pltpu.VMEM((NCAND,), jnp.int32),
            pltpu.VMEM((TPW * 16,), jnp.float32),
            pltpu.SemaphoreType.DMA,
        ],
    )(x2d, mxt.reshape(BH * C * NBLK), mnt.reshape(BH * C * NBLK),
      boff, pblk, poff)


@jax.jit
def kernel(inputs):
    xt = jnp.transpose(inputs, (0, 1, 3, 2))
    tables = _make_tables()
    halves = [_wave(xt, b0, tables) for b0 in range(0, B, BH)]
    return jnp.concatenate(halves).reshape(B, 2 * KK * C)


# speedup vs baseline: 2.0923x; 1.1311x over previous
import jax
import jax.numpy as jnp
import numpy as np
from jax import lax
from jax.experimental import pallas as pl
from jax.experimental.pallas import tpu as pltpu
from jax.experimental.pallas import tpu_sc as plsc

KK = 8
B, H, W, C = 8, 224, 224, 192
HW = H * W
G = 32
NJ = H // G
NBLK = NJ * W
NBLKP = 1600
L = 16
NCAND = KK * G
NW = 32
BH = 4
TPW = (BH * C) // NW
XCR = BH * H * C
HSTRIDE = C * 8
ROWS16 = 2 * XCR * 8

_NEG_INF = float("-inf")


def _stage1_body(x_ref, mx_ref, mn_ref, xc_ref):
    x = x_ref[...]
    mx_ref[...] = jnp.max(x, axis=1, keepdims=True)
    mn_ref[...] = jnp.min(x, axis=1, keepdims=True)
    xc_ref[0] = x[0, :, :, 0:128].reshape(G * C, 128)
    xc_ref[1] = x[0, :, :, 96:224].reshape(G * C, 128)


def _transpose_body(mx_ref, mn_ref, mxt_ref, mnt_ref):
    mxt_ref[...] = jnp.transpose(mx_ref[...], (0, 2, 1, 3))
    mnt_ref[...] = jnp.transpose(mn_ref[...], (0, 2, 1, 3))


def _mp(av, ai, bv, bi):
    sa, sai = plsc.sort_key_val(av, ai)
    sb, sbi = plsc.sort_key_val(bv, bi, descending=True)
    keep = sa >= sb
    return jnp.maximum(sa, sb), jnp.where(keep, sai, sbi)


def _scan_row(row_ref, nsteps4, negate):
    iota = lax.iota(jnp.int32, L)

    def step(i, carry):
        rv, ri = carry
        base = i * (4 * L)
        vs = []
        for k in range(4):
            v = row_ref[pl.ds(base + k * L, L)]
            if negate:
                v = -v
            vs.append((v, iota + (base + k * L)))
        ta, tai = _mp(vs[0][0], vs[0][1], vs[1][0], vs[1][1])
        tb, tbi = _mp(vs[2][0], vs[2][1], vs[3][0], vs[3][1])
        tt, tti = _mp(ta, tai, tb, tbi)
        nrv, nri = _mp(rv, ri, tt, tti)
        return nrv, nri

    rv0 = jnp.full((L,), _NEG_INF, jnp.float32)
    ri0 = jnp.zeros((L,), jnp.int32)
    rv, ri = lax.fori_loop(0, nsteps4, step, (rv0, ri0))
    srt = plsc.sort_key_val(rv, ri)
    return srt[0], srt[1]


def _sc_body(x2d, mxt, mnt, boff, pblk, poff, outf,
             mrow_v, nrow_v, boff_v, bb_v, bl_v, idx_v, rows_v,
             pblk_v, poff_v, out_v, sem):
    cid = lax.axis_index("c")
    sid = lax.axis_index("s")
    w = sid * 2 + cid
    b = w // 8
    c0 = 24 * (w % 8)
    iota = lax.iota(jnp.int32, L)

    pltpu.sync_copy(boff, boff_v)
    pltpu.sync_copy(pblk, pblk_v)
    pltpu.sync_copy(poff, poff_v)

    def task(tloc, _):
        c = c0 + tloc
        t = b * C + c
        pltpu.sync_copy(mxt.at[pl.ds(t * NBLK, NBLK)], mrow_v.at[pl.ds(0, NBLK)])
        pltpu.sync_copy(mnt.at[pl.ds(t * NBLK, NBLK)], nrow_v.at[pl.ds(0, NBLK)])
        for pk in range(NBLK, NBLKP, L):
            mrow_v[pl.ds(pk, L)] = jnp.full((L,), _NEG_INF, jnp.float32)
            nrow_v[pl.ds(pk, L)] = jnp.full((L,), -_NEG_INF, jnp.float32)
        rowbase = b * (H * C * 8) + c * 8

        def side(row_ref, negate, lane_off):
            rv, ri = _scan_row(row_ref, NBLKP // (4 * L), negate)
            bb_v[...] = plsc.load_gather(boff_v, [ri]) + rowbase
            bl_v[...] = jnp.bitwise_and(ri, L - 1)

            def build(i, _b):
                pb = pblk_v[pl.ds(i * L, L)]
                po = poff_v[pl.ds(i * L, L)]
                bv = plsc.load_gather(bb_v, [pb])
                idx_v[pl.ds(i * L, L)] = bv + po
                return 0

            lax.fori_loop(0, NCAND // L, build, 0)

            cp0 = pltpu.async_copy(x2d.at[idx_v.at[pl.ds(0, 128)]],
                                   rows_v.at[pl.ds(0, 128)], sem)
            cp1 = pltpu.async_copy(x2d.at[idx_v.at[pl.ds(128, 128)]],
                                   rows_v.at[pl.ds(128, 128)], sem)
            cp0.wait()
            cp1.wait()

            def cstep(i, rv2):
                base = i * (4 * L)
                cs = []
                for k in range(4):
                    pb = pblk_v[pl.ds(base + k * L, L)]
                    lv = plsc.load_gather(bl_v, [pb])
                    cv = plsc.load_gather(rows_v, [iota + (base + k * L), lv])
                    if negate:
                        cv = -cv
                    cs.append(cv)
                ta, _a = _mp(cs[0], cs[0], cs[1], cs[1])
                tb, _b = _mp(cs[2], cs[2], cs[3], cs[3])
                tt, _t = _mp(ta, ta, tb, tb)
                nv, _n = _mp(rv2, rv2, tt, tt)
                return nv

            rv2 = lax.fori_loop(0, NCAND // (4 * L), cstep,
                                jnp.full((L,), _NEG_INF, jnp.float32))
            rv2 = plsc.sort_key_val(rv2, rv2)[0]

            best_desc = lax.rev(rv2, (0,))
            if negate:
                vals = -best_desc
            else:
                vals = best_desc
            plsc.store_scatter(out_v, [iota + (16 * tloc + lane_off)], vals,
                               mask=iota < 8)

        side(mrow_v, False, 0)
        side(nrow_v, True, 8)
        return 0

    lax.fori_loop(0, TPW, task, 0)
    pltpu.sync_copy(out_v, outf.at[pl.ds(w * (TPW * 16), TPW * 16)])


def _make_tables():
    blk = np.arange(NBLK)
    jj, ww = blk // W, blk % W
    sel = (ww >= 128).astype(np.int64)
    wadj = ww - 96 * sel
    boff = (sel * (XCR * 8) + jj * G * HSTRIDE + wadj // L).astype(np.int32)
    j = np.arange(NCAND)
    pblk = (8 + j // G).astype(np.int32)
    poff = ((j % G) * HSTRIDE).astype(np.int32)
    return jnp.asarray(boff), jnp.asarray(pblk), jnp.asarray(poff)


def _wave(xt, b0, tables):
    boff, pblk, poff = tables

    mx, mn, xcopy = pl.pallas_call(
        _stage1_body,
        grid=(BH, NJ),
        in_specs=[pl.BlockSpec((1, G, C, W), lambda b, j: (b + b0, j, 0, 0))],
        out_specs=[
            pl.BlockSpec((1, 1, C, W), lambda b, j: (b, j, 0, 0)),
            pl.BlockSpec((1, 1, C, W), lambda b, j: (b, j, 0, 0)),
            pl.BlockSpec((2, G * C, 128), lambda b, j: (0, b * NJ + j, 0)),
        ],
        out_shape=[
            jax.ShapeDtypeStruct((BH, NJ, C, W), jnp.float32),
            jax.ShapeDtypeStruct((BH, NJ, C, W), jnp.float32),
            jax.ShapeDtypeStruct((2, XCR, 128), jnp.float32),
        ],
    )(xt)

    mxt, mnt = pl.pallas_call(
        _transpose_body,
        grid=(BH,),
        in_specs=[pl.BlockSpec((1, NJ, C, W), lambda b: (b, 0, 0, 0)),
                  pl.BlockSpec((1, NJ, C, W), lambda b: (b, 0, 0, 0))],
        out_specs=[pl.BlockSpec((1, C, NJ, W), lambda b: (b, 0, 0, 0)),
                   pl.BlockSpec((1, C, NJ, W), lambda b: (b, 0, 0, 0))],
        out_shape=[jax.ShapeDtypeStruct((BH, C, NJ, W), jnp.float32)] * 2,
    )(mx, mn)

    x2d = xcopy.reshape(ROWS16, L)

    mesh = plsc.VectorSubcoreMesh(core_axis_name="c", subcore_axis_name="s",
                                  num_cores=2, num_subcores=16)
    return pl.kernel(
        _sc_body,
        out_type=jax.ShapeDtypeStruct((BH * C * 2 * KK,), jnp.float32),
        mesh=mesh,
        compiler_params=pltpu.CompilerParams(needs_layout_passes=False,
                                             use_tc_tiling_on_sc=False),
        scratch_types=[
            pltpu.VMEM((NBLKP,), jnp.float32),
            pltpu.VMEM((NBLKP,), jnp.float32),
            pltpu.VMEM((NBLK,), jnp.int32),
            pltpu.VMEM((L,), jnp.int32),
            pltpu.VMEM((L,), jnp.int32),
            pltpu.VMEM((NCAND,), jnp.int32),
            pltpu.VMEM((NCAND, L), jnp.float32),
            pltpu.VMEM((NCAND,), jnp.int32),
            pltpu.VMEM((NCAND,), jnp.int32),
            pltpu.VMEM((TPW * 16,), jnp.float32),
            pltpu.SemaphoreType.DMA,
        ],
    )(x2d, mxt.reshape(BH * C * NBLK), mnt.reshape(BH * C * NBLK),
      boff, pblk, poff)


@jax.jit
def kernel(inputs):
    xt = jnp.transpose(inputs, (0, 1, 3, 2))
    tables = _make_tables()
    halves = [_wave(xt, b0, tables) for b0 in range(0, B, BH)]
    return jnp.concatenate(halves).reshape(B, 2 * KK * C)


# speedup vs baseline: 2.1608x; 1.0327x over previous
import jax
import jax.numpy as jnp
import numpy as np
from jax import lax
from jax.experimental import pallas as pl
from jax.experimental.pallas import tpu as pltpu
from jax.experimental.pallas import tpu_sc as plsc

KK = 8
B, H, W, C = 8, 224, 224, 192
HW = H * W
G = 32
NJ = H // G
NBLK = NJ * W
NBLKP = 1600
L = 16
NCAND = KK * G
NW = 32
BH = 2
TPW = (BH * C) // NW
WPB = NW // BH
XCR = BH * H * C
HSTRIDE = C * 8
ROWS16 = 2 * XCR * 8

_NEG_INF = float("-inf")


def _stage1_body(x_ref, mx_ref, mn_ref, xc_ref):
    x = x_ref[...]
    mx_ref[...] = jnp.max(x, axis=1, keepdims=True)
    mn_ref[...] = jnp.min(x, axis=1, keepdims=True)
    xc_ref[0] = x[0, :, :, 0:128].reshape(G * C, 128)
    xc_ref[1] = x[0, :, :, 96:224].reshape(G * C, 128)


def _transpose_body(mx_ref, mn_ref, mxt_ref, mnt_ref):
    mxt_ref[...] = jnp.transpose(mx_ref[...], (0, 2, 1, 3))
    mnt_ref[...] = jnp.transpose(mn_ref[...], (0, 2, 1, 3))


def _mp(av, ai, bv, bi):
    sa, sai = plsc.sort_key_val(av, ai)
    sb, sbi = plsc.sort_key_val(bv, bi, descending=True)
    keep = sa >= sb
    return jnp.maximum(sa, sb), jnp.where(keep, sai, sbi)


def _scan_row(row_ref, nsteps4, negate):
    iota = lax.iota(jnp.int32, L)

    def step(i, carry):
        rv, ri = carry
        base = i * (4 * L)
        vs = []
        for k in range(4):
            v = row_ref[pl.ds(base + k * L, L)]
            if negate:
                v = -v
            vs.append((v, iota + (base + k * L)))
        ta, tai = _mp(vs[0][0], vs[0][1], vs[1][0], vs[1][1])
        tb, tbi = _mp(vs[2][0], vs[2][1], vs[3][0], vs[3][1])
        tt, tti = _mp(ta, tai, tb, tbi)
        nrv, nri = _mp(rv, ri, tt, tti)
        return nrv, nri

    rv0 = jnp.full((L,), _NEG_INF, jnp.float32)
    ri0 = jnp.zeros((L,), jnp.int32)
    rv, ri = lax.fori_loop(0, nsteps4, step, (rv0, ri0))
    srt = plsc.sort_key_val(rv, ri)
    return srt[0], srt[1]


def _sc_body(x2d, mxt, mnt, boff, pblk, poff, outf,
             mrow_v, nrow_v, boff_v, bb_v, bl_v, idx_v, rows_v,
             pblk_v, poff_v, out_v, sem):
    cid = lax.axis_index("c")
    sid = lax.axis_index("s")
    w = sid * 2 + cid
    b = w // WPB
    c0 = TPW * (w % WPB)
    iota = lax.iota(jnp.int32, L)

    pltpu.sync_copy(boff, boff_v)
    pltpu.sync_copy(pblk, pblk_v)
    pltpu.sync_copy(poff, poff_v)

    def task(tloc, _):
        c = c0 + tloc
        t = b * C + c
        pltpu.sync_copy(mxt.at[pl.ds(t * NBLK, NBLK)], mrow_v.at[pl.ds(0, NBLK)])
        pltpu.sync_copy(mnt.at[pl.ds(t * NBLK, NBLK)], nrow_v.at[pl.ds(0, NBLK)])
        for pk in range(NBLK, NBLKP, L):
            mrow_v[pl.ds(pk, L)] = jnp.full((L,), _NEG_INF, jnp.float32)
            nrow_v[pl.ds(pk, L)] = jnp.full((L,), -_NEG_INF, jnp.float32)
        rowbase = b * (H * C * 8) + c * 8

        def side(row_ref, negate, lane_off):
            rv, ri = _scan_row(row_ref, NBLKP // (4 * L), negate)
            bb_v[...] = plsc.load_gather(boff_v, [ri]) + rowbase
            bl_v[...] = jnp.bitwise_and(ri, L - 1)

            def build(i, _b):
                pb = pblk_v[pl.ds(i * L, L)]
                po = poff_v[pl.ds(i * L, L)]
                bv = plsc.load_gather(bb_v, [pb])
                idx_v[pl.ds(i * L, L)] = bv + po
                return 0

            lax.fori_loop(0, NCAND // L, build, 0)

            cp0 = pltpu.async_copy(x2d.at[idx_v.at[pl.ds(0, 128)]],
                                   rows_v.at[pl.ds(0, 128)], sem)
            cp1 = pltpu.async_copy(x2d.at[idx_v.at[pl.ds(128, 128)]],
                                   rows_v.at[pl.ds(128, 128)], sem)
            cp0.wait()
            cp1.wait()

            def cstep(i, rv2):
                base = i * (4 * L)
                cs = []
                for k in range(4):
                    pb = pblk_v[pl.ds(base + k * L, L)]
                    lv = plsc.load_gather(bl_v, [pb])
                    cv = plsc.load_gather(rows_v, [iota + (base + k * L), lv])
                    if negate:
                        cv = -cv
                    cs.append(cv)
                ta, _a = _mp(cs[0], cs[0], cs[1], cs[1])
                tb, _b = _mp(cs[2], cs[2], cs[3], cs[3])
                tt, _t = _mp(ta, ta, tb, tb)
                nv, _n = _mp(rv2, rv2, tt, tt)
                return nv

            rv2 = lax.fori_loop(0, NCAND // (4 * L), cstep,
                                jnp.full((L,), _NEG_INF, jnp.float32))
            rv2 = plsc.sort_key_val(rv2, rv2)[0]

            best_desc = lax.rev(rv2, (0,))
            if negate:
                vals = -best_desc
            else:
                vals = best_desc
            plsc.store_scatter(out_v, [iota + (16 * tloc + lane_off)], vals,
                               mask=iota < 8)

        side(mrow_v, False, 0)
        side(nrow_v, True, 8)
        return 0

    lax.fori_loop(0, TPW, task, 0)
    pltpu.sync_copy(out_v, outf.at[pl.ds(w * (TPW * 16), TPW * 16)])


def _make_tables():
    blk = np.arange(NBLK)
    jj, ww = blk // W, blk % W
    sel = (ww >= 128).astype(np.int64)
    wadj = ww - 96 * sel
    boff = (sel * (XCR * 8) + jj * G * HSTRIDE + wadj // L).astype(np.int32)
    j = np.arange(NCAND)
    pblk = (8 + j // G).astype(np.int32)
    poff = ((j % G) * HSTRIDE).astype(np.int32)
    return jnp.asarray(boff), jnp.asarray(pblk), jnp.asarray(poff)


def _wave(xt, b0, tables):
    boff, pblk, poff = tables

    mx, mn, xcopy = pl.pallas_call(
        _stage1_body,
        grid=(BH, NJ),
        in_specs=[pl.BlockSpec((1, G, C, W), lambda b, j: (b + b0, j, 0, 0))],
        out_specs=[
            pl.BlockSpec((1, 1, C, W), lambda b, j: (b, j, 0, 0)),
            pl.BlockSpec((1, 1, C, W), lambda b, j: (b, j, 0, 0)),
            pl.BlockSpec((2, G * C, 128), lambda b, j: (0, b * NJ + j, 0)),
        ],
        out_shape=[
            jax.ShapeDtypeStruct((BH, NJ, C, W), jnp.float32),
            jax.ShapeDtypeStruct((BH, NJ, C, W), jnp.float32),
            jax.ShapeDtypeStruct((2, XCR, 128), jnp.float32),
        ],
    )(xt)

    mxt, mnt = pl.pallas_call(
        _transpose_body,
        grid=(BH,),
        in_specs=[pl.BlockSpec((1, NJ, C, W), lambda b: (b, 0, 0, 0)),
                  pl.BlockSpec((1, NJ, C, W), lambda b: (b, 0, 0, 0))],
        out_specs=[pl.BlockSpec((1, C, NJ, W), lambda b: (b, 0, 0, 0)),
                   pl.BlockSpec((1, C, NJ, W), lambda b: (b, 0, 0, 0))],
        out_shape=[jax.ShapeDtypeStruct((BH, C, NJ, W), jnp.float32)] * 2,
    )(mx, mn)

    x2d = xcopy.reshape(ROWS16, L)

    mesh = plsc.VectorSubcoreMesh(core_axis_name="c", subcore_axis_name="s",
                                  num_cores=2, num_subcores=16)
    return pl.kernel(
        _sc_body,
        out_type=jax.ShapeDtypeStruct((BH * C * 2 * KK,), jnp.float32),
        mesh=mesh,
        compiler_params=pltpu.CompilerParams(needs_layout_passes=False,
                                             use_tc_tiling_on_sc=False),
        scratch_types=[
            pltpu.VMEM((NBLKP,), jnp.float32),
            pltpu.VMEM((NBLKP,), jnp.float32),
            pltpu.VMEM((NBLK,), jnp.int32),
            pltpu.VMEM((L,), jnp.int32),
            pltpu.VMEM((L,), jnp.int32),
            pltpu.VMEM((NCAND,), jnp.int32),
            pltpu.VMEM((NCAND, L), jnp.float32),
            pltpu.VMEM((NCAND,), jnp.int32),
            pltpu.VMEM((NCAND,), jnp.int32),
            pltpu.VMEM((TPW * 16,), jnp.float32),
            pltpu.SemaphoreType.DMA,
        ],
    )(x2d, mxt.reshape(BH * C * NBLK), mnt.reshape(BH * C * NBLK),
      boff, pblk, poff)


@jax.jit
def kernel(inputs):
    xt = jnp.transpose(inputs, (0, 1, 3, 2))
    tables = _make_tables()
    halves = [_wave(xt, b0, tables) for b0 in range(0, B, BH)]
    return jnp.concatenate(halves).reshape(B, 2 * KK * C)


# speedup vs baseline: 2.5592x; 1.1844x over previous
import jax
import jax.numpy as jnp
import numpy as np
from jax import lax
from jax.experimental import pallas as pl
from jax.experimental.pallas import tpu as pltpu
from jax.experimental.pallas import tpu_sc as plsc

KK = 8
B, H, W, C = 8, 224, 224, 192
HW = H * W
G = 32
NJ = H // G
NBLK = NJ * W
NBLKP = 1600
L = 16
NCAND = KK * G
NW = 32
BH = 2
TPW = (BH * C) // NW
WPB = NW // BH
XCR = BH * H * C
HSTRIDE = C * 8
ROWS16 = 2 * XCR * 8

_NEG_INF = float("-inf")


def _stage1_body(x_ref, mx_ref, mn_ref, xc_ref):
    x = x_ref[...]
    mx_ref[...] = jnp.max(x, axis=1, keepdims=True)
    mn_ref[...] = jnp.min(x, axis=1, keepdims=True)
    xc_ref[0] = x[0, :, :, 0:128].reshape(G * C, 128)
    xc_ref[1] = x[0, :, :, 96:224].reshape(G * C, 128)


def _transpose_body(mx_ref, mn_ref, mxt_ref, mnt_ref):
    mxt_ref[...] = jnp.transpose(mx_ref[...], (0, 2, 1, 3))
    mnt_ref[...] = jnp.transpose(mn_ref[...], (0, 2, 1, 3))


def _mp(av, ai, bv, bi):
    sa, sai = plsc.sort_key_val(av, ai)
    sb, sbi = plsc.sort_key_val(bv, bi, descending=True)
    keep = sa >= sb
    return jnp.maximum(sa, sb), jnp.where(keep, sai, sbi)


def _scan_row(row_ref, nsteps4, negate):
    iota = lax.iota(jnp.int32, L)

    def step(i, carry):
        rv, ri = carry
        base = i * (4 * L)
        vs = []
        for k in range(4):
            v = row_ref[pl.ds(base + k * L, L)]
            if negate:
                v = -v
            vs.append((v, iota + (base + k * L)))
        ta, tai = _mp(vs[0][0], vs[0][1], vs[1][0], vs[1][1])
        tb, tbi = _mp(vs[2][0], vs[2][1], vs[3][0], vs[3][1])
        tt, tti = _mp(ta, tai, tb, tbi)
        nrv, nri = _mp(rv, ri, tt, tti)
        return nrv, nri

    rv0 = jnp.full((L,), _NEG_INF, jnp.float32)
    ri0 = jnp.zeros((L,), jnp.int32)
    rv, ri = lax.fori_loop(0, nsteps4, step, (rv0, ri0))
    srt = plsc.sort_key_val(rv, ri)
    return srt[0], srt[1]


def _sc_body(x2d, mxt, mnt, boff, pblk, poff, outf,
             mrow_a, nrow_a, mrow_b, nrow_b, boff_v,
             bb_t, bl_t, bb_u, bl_u, idx_t, idx_u, rows_t, rows_u,
             pblk_v, poff_v, out_v, sem_a, sem_b, sem_g, sem_h):
    cid = lax.axis_index("c")
    sid = lax.axis_index("s")
    w = sid * 2 + cid
    b = w // WPB
    c0 = TPW * (w % WPB)
    iota = lax.iota(jnp.int32, L)

    pltpu.sync_copy(boff, boff_v)
    pltpu.sync_copy(pblk, pblk_v)
    pltpu.sync_copy(poff, poff_v)
    for buf, pad in ((mrow_a, _NEG_INF), (nrow_a, -_NEG_INF),
                     (mrow_b, _NEG_INF), (nrow_b, -_NEG_INF)):
        for pk in range(NBLK, NBLKP, L):
            buf[pl.ds(pk, L)] = jnp.full((L,), pad, jnp.float32)

    def fire_rows(tloc, bm, bn, sem):
        t = b * C + c0 + tloc
        pltpu.async_copy(mxt.at[pl.ds(t * NBLK, NBLK)],
                         bm.at[pl.ds(0, NBLK)], sem)
        pltpu.async_copy(mnt.at[pl.ds(t * NBLK, NBLK)],
                         bn.at[pl.ds(0, NBLK)], sem)

    def wait_rows(bm, bn, sem):
        pltpu.make_async_copy(mxt.at[pl.ds(0, NBLK)],
                              bm.at[pl.ds(0, NBLK)], sem).wait()
        pltpu.make_async_copy(mnt.at[pl.ds(0, NBLK)],
                              bn.at[pl.ds(0, NBLK)], sem).wait()

    def scan_and_fire(row_ref, negate, rowbase, bb_v, bl_v, idx_v, rows_v,
                      sem):
        rv, ri = _scan_row(row_ref, NBLKP // (4 * L), negate)
        bb_v[...] = plsc.load_gather(boff_v, [ri]) + rowbase
        bl_v[...] = jnp.bitwise_and(ri, L - 1)

        def build(i, _b):
            pb = pblk_v[pl.ds(i * L, L)]
            po = poff_v[pl.ds(i * L, L)]
            bv = plsc.load_gather(bb_v, [pb])
            idx_v[pl.ds(i * L, L)] = bv + po
            return 0

        lax.fori_loop(0, NCAND // L, build, 0)
        return [pltpu.async_copy(x2d.at[idx_v.at[pl.ds(k * 128, 128)]],
                                 rows_v.at[pl.ds(k * 128, 128)], sem)
                for k in range(NCAND // 128)]

    def consume(cps, negate, bl_v, rows_v, tloc, lane_off):
        for cp in cps:
            cp.wait()

        def cstep(i, rv2):
            base = i * (4 * L)
            cs = []
            for k in range(4):
                pb = pblk_v[pl.ds(base + k * L, L)]
                lv = plsc.load_gather(bl_v, [pb])
                cv = plsc.load_gather(rows_v, [iota + (base + k * L), lv])
                if negate:
                    cv = -cv
                cs.append(cv)
            ta, _a = _mp(cs[0], cs[0], cs[1], cs[1])
            tb, _b = _mp(cs[2], cs[2], cs[3], cs[3])
            tt, _t = _mp(ta, ta, tb, tb)
            nv, _n = _mp(rv2, rv2, tt, tt)
            return nv

        rv2 = lax.fori_loop(0, NCAND // (4 * L), cstep,
                            jnp.full((L,), _NEG_INF, jnp.float32))
        rv2 = plsc.sort_key_val(rv2, rv2)[0]
        best_desc = lax.rev(rv2, (0,))
        vals = -best_desc if negate else best_desc
        plsc.store_scatter(out_v, [iota + (16 * tloc + lane_off)], vals,
                           mask=iota < 8)

    def process(tloc, bm, bn):
        c = c0 + tloc
        rowbase = b * (H * C * 8) + c * 8
        cps_t = scan_and_fire(bm, False, rowbase, bb_t, bl_t, idx_t, rows_t,
                              sem_g)
        cps_u = scan_and_fire(bn, True, rowbase, bb_u, bl_u, idx_u, rows_u,
                              sem_h)
        consume(cps_t, False, bl_t, rows_t, tloc, 0)
        consume(cps_u, True, bl_u, rows_u, tloc, 8)

    fire_rows(0, mrow_a, nrow_a, sem_a)

    def pair(i, _):
        te = 2 * i
        fire_rows(te + 1, mrow_b, nrow_b, sem_b)
        wait_rows(mrow_a, nrow_a, sem_a)
        process(te, mrow_a, nrow_a)
        fire_rows(jnp.minimum(te + 2, TPW - 1), mrow_a, nrow_a, sem_a)
        wait_rows(mrow_b, nrow_b, sem_b)
        process(te + 1, mrow_b, nrow_b)
        return 0

    lax.fori_loop(0, TPW // 2, pair, 0)
    wait_rows(mrow_a, nrow_a, sem_a)
    pltpu.sync_copy(out_v, outf.at[pl.ds(w * (TPW * 16), TPW * 16)])


def _make_tables():
    blk = np.arange(NBLK)
    jj, ww = blk // W, blk % W
    sel = (ww >= 128).astype(np.int64)
    wadj = ww - 96 * sel
    boff = (sel * (XCR * 8) + jj * G * HSTRIDE + wadj // L).astype(np.int32)
    j = np.arange(NCAND)
    pblk = (8 + j // G).astype(np.int32)
    poff = ((j % G) * HSTRIDE).astype(np.int32)
    return jnp.asarray(boff), jnp.asarray(pblk), jnp.asarray(poff)


def _wave(xt, b0, tables):
    boff, pblk, poff = tables

    mx, mn, xcopy = pl.pallas_call(
        _stage1_body,
        grid=(BH, NJ),
        in_specs=[pl.BlockSpec((1, G, C, W), lambda b, j: (b + b0, j, 0, 0))],
        out_specs=[
            pl.BlockSpec((1, 1, C, W), lambda b, j: (b, j, 0, 0)),
            pl.BlockSpec((1, 1, C, W), lambda b, j: (b, j, 0, 0)),
            pl.BlockSpec((2, G * C, 128), lambda b, j: (0, b * NJ + j, 0)),
        ],
        out_shape=[
            jax.ShapeDtypeStruct((BH, NJ, C, W), jnp.float32),
            jax.ShapeDtypeStruct((BH, NJ, C, W), jnp.float32),
            jax.ShapeDtypeStruct((2, XCR, 128), jnp.float32),
        ],
    )(xt)

    mxt, mnt = pl.pallas_call(
        _transpose_body,
        grid=(BH,),
        in_specs=[pl.BlockSpec((1, NJ, C, W), lambda b: (b, 0, 0, 0)),
                  pl.BlockSpec((1, NJ, C, W), lambda b: (b, 0, 0, 0))],
        out_specs=[pl.BlockSpec((1, C, NJ, W), lambda b: (b, 0, 0, 0)),
                   pl.BlockSpec((1, C, NJ, W), lambda b: (b, 0, 0, 0))],
        out_shape=[jax.ShapeDtypeStruct((BH, C, NJ, W), jnp.float32)] * 2,
    )(mx, mn)

    x2d = xcopy.reshape(ROWS16, L)

    mesh = plsc.VectorSubcoreMesh(core_axis_name="c", subcore_axis_name="s",
                                  num_cores=2, num_subcores=16)
    return pl.kernel(
        _sc_body,
        out_type=jax.ShapeDtypeStruct((BH * C * 2 * KK,), jnp.float32),
        mesh=mesh,
        compiler_params=pltpu.CompilerParams(needs_layout_passes=False,
                                             use_tc_tiling_on_sc=False),
        scratch_types=[
            pltpu.VMEM((NBLKP,), jnp.float32),
            pltpu.VMEM((NBLKP,), jnp.float32),
            pltpu.VMEM((NBLKP,), jnp.float32),
            pltpu.VMEM((NBLKP,), jnp.float32),
            pltpu.VMEM((NBLK,), jnp.int32),
            pltpu.VMEM((L,), jnp.int32),
            pltpu.VMEM((L,), jnp.int32),
            pltpu.VMEM((L,), jnp.int32),
            pltpu.VMEM((L,), jnp.int32),
            pltpu.VMEM((NCAND,), jnp.int32),
            pltpu.VMEM((NCAND,), jnp.int32),
            pltpu.VMEM((NCAND, L), jnp.float32),
            pltpu.VMEM((NCAND, L), jnp.float32),
            pltpu.VMEM((NCAND,), jnp.int32),
            pltpu.VMEM((NCAND,), jnp.int32),
            pltpu.VMEM((TPW * 16,), jnp.float32),
            pltpu.SemaphoreType.DMA,
            pltpu.SemaphoreType.DMA,
            pltpu.SemaphoreType.DMA,
            pltpu.SemaphoreType.DMA,
        ],
    )(x2d, mxt.reshape(BH * C * NBLK), mnt.reshape(BH * C * NBLK),
      boff, pblk, poff)


@jax.jit
def kernel(inputs):
    xt = jnp.transpose(inputs, (0, 1, 3, 2))
    tables = _make_tables()
    halves = [_wave(xt, b0, tables) for b0 in range(0, B, BH)]
    return jnp.concatenate(halves).reshape(B, 2 * KK * C)
